# Initial kernel scaffold; baseline (speedup 1.0000x reference)
import jax
import jax.numpy as jnp
from jax.experimental import pallas as pl


def kernel(x, w0, b0, w1, b1, w2, b2, w3, b3, w4, b4):
    raise NotImplementedError("write your pallas kernel here")


# speedup vs baseline: 1.0019x; 1.0019x over previous
import jax
import jax.numpy as jnp
from jax.experimental import pallas as pl
from jax.experimental.pallas import tpu as pltpu

_TM = 2048


def _siren_kernel(x_ref, w0_ref, b0_ref, w1_ref, b1_ref, w2_ref, b2_ref,
                  w3_ref, b3_ref, w4_ref, b4_ref, o_ref):
    act = jnp.sin(
        jnp.dot(x_ref[...], w0_ref[...], preferred_element_type=jnp.float32)
        + b0_ref[...])
    for w_ref, b_ref in ((w1_ref, b1_ref), (w2_ref, b2_ref), (w3_ref, b3_ref)):
        z = jnp.dot(act, w_ref[...], preferred_element_type=jnp.float32)
        act = jnp.sin(z + b_ref[...])
    z4 = jnp.dot(act, w4_ref[...], preferred_element_type=jnp.float32)
    o_ref[...] = jnp.sin(z4[:, :3] + b4_ref[0, :3])


def kernel(x, w0, b0, w1, b1, w2, b2, w3, b3, w4, b4):
    n, in_f = x.shape
    grid = (pl.cdiv(n, _TM),)

    def _resident(shape):
        return pl.BlockSpec(shape, lambda i: (0,) * len(shape))

    weight_args = [w0, b0, w1, b1, w2, b2, w3, b3, w4, b4]
    in_specs = [pl.BlockSpec((_TM, in_f), lambda i: (i, 0))]
    in_specs += [_resident(a.shape) for a in weight_args]

    flops = 2 * n * (in_f * 256 + 3 * 256 * 256 + 256 * 128)
    cost = pl.CostEstimate(
        flops=flops,
        transcendentals=n * (4 * 256 + 3),
        bytes_accessed=n * in_f * 4 + n * 3 * 4 + sum(
            a.size * 4 for a in weight_args),
    )

    return pl.pallas_call(
        _siren_kernel,
        out_shape=jax.ShapeDtypeStruct((n, 3), jnp.float32),
        grid=grid,
        in_specs=in_specs,
        out_specs=pl.BlockSpec((_TM, 3), lambda i: (i, 0)),
        compiler_params=pltpu.CompilerParams(
            dimension_semantics=("parallel",),
            vmem_limit_bytes=56 << 20,
        ),
        cost_estimate=cost,
    )(x, *weight_args)


# speedup vs baseline: 5.5275x; 5.5171x over previous
import jax
import jax.numpy as jnp
from jax.experimental import pallas as pl
from jax.experimental.pallas import tpu as pltpu

_TM = 2048

_INV_2PI = 0.15915494309189535
_MAGIC = 1.5 * 2.0**23

_C1 = 6.283054828643799
_C3 = -41.33115005493164
_C5 = 81.36590576171875
_C7 = -74.47315979003906
_C9 = 32.772769927978516


def _sin_2pi(zt):
    t = zt + _MAGIC
    k = t - _MAGIC
    u = zt - k
    u2 = u * u
    p = _C9
    for c in (_C7, _C5, _C3, _C1):
        p = p * u2 + c
    return p * u


_C = float(_INV_2PI)


def _siren_kernel(x_ref, w0_ref, b0_ref, w1_ref, b1_ref, w2_ref, b2_ref,
                  w3_ref, b3_ref, w4_ref, b4_ref, o_ref):
    act = _sin_2pi(
        jnp.dot(x_ref[...], w0_ref[...], preferred_element_type=jnp.float32)
        * _C + b0_ref[...])
    for w_ref, b_ref in ((w1_ref, b1_ref), (w2_ref, b2_ref), (w3_ref, b3_ref)):
        z = jnp.dot(act, w_ref[...], preferred_element_type=jnp.float32)
        act = _sin_2pi(z * _C + b_ref[...])
    z4 = jnp.dot(act, w4_ref[...], preferred_element_type=jnp.float32)
    o_ref[...] = _sin_2pi(z4[:, :3] * _C + b4_ref[0, :3])


def kernel(x, w0, b0, w1, b1, w2, b2, w3, b3, w4, b4):
    n, in_f = x.shape
    grid = (pl.cdiv(n, _TM),)

    def _resident(shape):
        return pl.BlockSpec(shape, lambda i: (0,) * len(shape))

    c = jnp.float32(_INV_2PI)
    scaled = [w0, b0 * c, w1, b1 * c, w2, b2 * c, w3, b3 * c, w4, b4 * c]
    in_specs = [pl.BlockSpec((_TM, in_f), lambda i: (i, 0))]
    in_specs += [_resident(a.shape) for a in scaled]

    flops = 2 * n * (in_f * 256 + 3 * 256 * 256 + 256 * 128)
    cost = pl.CostEstimate(
        flops=flops,
        transcendentals=n * (4 * 256 + 3),
        bytes_accessed=n * in_f * 4 + n * 3 * 4 + sum(
            a.size * 4 for a in scaled),
    )

    return pl.pallas_call(
        _siren_kernel,
        out_shape=jax.ShapeDtypeStruct((n, 3), jnp.float32),
        grid=grid,
        in_specs=in_specs,
        out_specs=pl.BlockSpec((_TM, 3), lambda i: (i, 0)),
        compiler_params=pltpu.CompilerParams(
            dimension_semantics=("parallel",),
            vmem_limit_bytes=56 << 20,
        ),
        cost_estimate=cost,
    )(x, *scaled)


# speedup vs baseline: 5.9699x; 1.0800x over previous
import jax
import jax.numpy as jnp
from jax.experimental import pallas as pl
from jax.experimental.pallas import tpu as pltpu

_TM = 4096

_INV_2PI = 0.15915494309189535
_A = 1.3

_P0 = 4.833189964294434
_P1 = -29.079370498657227
_P2 = 44.6170768737793
_P3 = -22.523578643798828


def _sin_2pi(zt):
    k = jnp.rint(zt)
    u = zt - k
    t = jnp.tanh(_A * u)
    u2 = u * u
    p = (_P3 * u2 + _P2) * u2 + _P1
    p = p * u2 + _P0
    return t * p


_C = float(_INV_2PI)


def _siren_kernel(x_ref, w0_ref, b0_ref, w1_ref, b1_ref, w2_ref, b2_ref,
                  w3_ref, b3_ref, w4_ref, b4_ref, o_ref):
    act = _sin_2pi(
        jnp.dot(x_ref[...], w0_ref[...], preferred_element_type=jnp.float32)
        * _C + b0_ref[...])
    for w_ref, b_ref in ((w1_ref, b1_ref), (w2_ref, b2_ref), (w3_ref, b3_ref)):
        z = jnp.dot(act, w_ref[...], preferred_element_type=jnp.float32)
        act = _sin_2pi(z * _C + b_ref[...])
    z4 = jnp.dot(act, w4_ref[...], preferred_element_type=jnp.float32)
    o_ref[...] = _sin_2pi(z4[:, :3] * _C + b4_ref[0, :3])


def kernel(x, w0, b0, w1, b1, w2, b2, w3, b3, w4, b4):
    n, in_f = x.shape
    grid = (pl.cdiv(n, _TM),)

    def _resident(shape):
        return pl.BlockSpec(shape, lambda i: (0,) * len(shape))

    c = jnp.float32(_INV_2PI)
    scaled = [w0, b0 * c, w1, b1 * c, w2, b2 * c, w3, b3 * c, w4, b4 * c]
    in_specs = [pl.BlockSpec((_TM, in_f), lambda i: (i, 0))]
    in_specs += [_resident(a.shape) for a in scaled]

    flops = 2 * n * (in_f * 256 + 3 * 256 * 256 + 256 * 128)
    cost = pl.CostEstimate(
        flops=flops,
        transcendentals=n * (4 * 256 + 3),
        bytes_accessed=n * in_f * 4 + n * 3 * 4 + sum(
            a.size * 4 for a in scaled),
    )

    return pl.pallas_call(
        _siren_kernel,
        out_shape=jax.ShapeDtypeStruct((n, 3), jnp.float32),
        grid=grid,
        in_specs=in_specs,
        out_specs=pl.BlockSpec((_TM, 3), lambda i: (i, 0)),
        compiler_params=pltpu.CompilerParams(
            dimension_semantics=("parallel",),
            vmem_limit_bytes=56 << 20,
        ),
        cost_estimate=cost,
    )(x, *scaled)


# speedup vs baseline: 6.0370x; 1.0112x over previous
import jax
import jax.numpy as jnp
from jax.experimental import pallas as pl
from jax.experimental.pallas import tpu as pltpu

_TM = 8192

_INV_2PI = 0.15915494309189535
_A = 1.3

_P0 = 4.833189964294434
_P1 = -29.079370498657227
_P2 = 44.6170768737793
_P3 = -22.523578643798828


def _sin_2pi(zt):
    k = jnp.rint(zt)
    u = zt - k
    t = jnp.tanh(_A * u)
    u2 = u * u
    p = (_P3 * u2 + _P2) * u2 + _P1
    p = p * u2 + _P0
    return t * p


_C = float(_INV_2PI)


def _siren_kernel(x_ref, w0_ref, b0_ref, w1_ref, b1_ref, w2_ref, b2_ref,
                  w3_ref, b3_ref, w4_ref, b4_ref, o_ref):
    act = _sin_2pi(
        jnp.dot(x_ref[...], w0_ref[...], preferred_element_type=jnp.float32)
        * _C + b0_ref[...])
    for w_ref, b_ref in ((w1_ref, b1_ref), (w2_ref, b2_ref), (w3_ref, b3_ref)):
        z = jnp.dot(act, w_ref[...], preferred_element_type=jnp.float32)
        act = _sin_2pi(z * _C + b_ref[...])
    z4 = jnp.dot(act, w4_ref[...], preferred_element_type=jnp.float32)
    o_ref[...] = _sin_2pi(z4[:, :3] * _C + b4_ref[0, :3])


def kernel(x, w0, b0, w1, b1, w2, b2, w3, b3, w4, b4):
    n, in_f = x.shape
    grid = (pl.cdiv(n, _TM),)

    def _resident(shape):
        return pl.BlockSpec(shape, lambda i: (0,) * len(shape))

    c = jnp.float32(_INV_2PI)
    scaled = [w0, b0 * c, w1, b1 * c, w2, b2 * c, w3, b3 * c, w4, b4 * c]
    in_specs = [pl.BlockSpec((_TM, in_f), lambda i: (i, 0))]
    in_specs += [_resident(a.shape) for a in scaled]

    flops = 2 * n * (in_f * 256 + 3 * 256 * 256 + 256 * 128)
    cost = pl.CostEstimate(
        flops=flops,
        transcendentals=n * (4 * 256 + 3),
        bytes_accessed=n * in_f * 4 + n * 3 * 4 + sum(
            a.size * 4 for a in scaled),
    )

    return pl.pallas_call(
        _siren_kernel,
        out_shape=jax.ShapeDtypeStruct((n, 3), jnp.float32),
        grid=grid,
        in_specs=in_specs,
        out_specs=pl.BlockSpec((_TM, 3), lambda i: (i, 0)),
        compiler_params=pltpu.CompilerParams(
            dimension_semantics=("parallel",),
            vmem_limit_bytes=56 << 20,
        ),
        cost_estimate=cost,
    )(x, *scaled)


# speedup vs baseline: 7.3220x; 1.2129x over previous
import jax
import jax.numpy as jnp
from jax.experimental import pallas as pl
from jax.experimental.pallas import tpu as pltpu

_TM = 8192

_INV_2PI = 0.15915494309189535
_A = 1.3

_P0 = 4.833189964294434
_P1 = -29.079370498657227
_P2 = 44.6170768737793
_P3 = -22.523578643798828


def _sin_2pi(zt):
    k = jnp.rint(zt)
    u = zt - k
    t = jnp.tanh(_A * u)
    u2 = u * u
    p = (_P3 * u2 + _P2) * u2 + _P1
    p = p * u2 + _P0
    return t * p


_C = float(_INV_2PI)


def _siren_kernel(x_ref, w0_ref, b0_ref, w1_ref, b1_ref, w2_ref, b2_ref,
                  w3_ref, b3_ref, w4_ref, b4_ref, o_ref):
    act = _sin_2pi(
        jnp.dot(x_ref[...], w0_ref[...], preferred_element_type=jnp.float32)
        * _C + b0_ref[...])
    for w_ref, b_ref in ((w1_ref, b1_ref), (w2_ref, b2_ref), (w3_ref, b3_ref)):
        z = jnp.dot(act, w_ref[...], preferred_element_type=jnp.float32)
        act = _sin_2pi(z * _C + b_ref[...])
    z4 = jnp.dot(act, w4_ref[...], preferred_element_type=jnp.float32)
    z4t = jnp.swapaxes(z4, 0, 1)[:3, :]
    o_ref[...] = _sin_2pi(z4t * _C + b4_ref[...])


def kernel(x, w0, b0, w1, b1, w2, b2, w3, b3, w4, b4):
    n, in_f = x.shape
    grid = (pl.cdiv(n, _TM),)

    def _resident(shape):
        return pl.BlockSpec(shape, lambda i: (0,) * len(shape))

    c = jnp.float32(_INV_2PI)
    b4t = jnp.transpose(b4 * c)[:3, :]
    scaled = [w0, b0 * c, w1, b1 * c, w2, b2 * c, w3, b3 * c, w4, b4t]
    in_specs = [pl.BlockSpec((_TM, in_f), lambda i: (i, 0))]
    in_specs += [_resident(a.shape) for a in scaled]

    flops = 2 * n * (in_f * 256 + 3 * 256 * 256 + 256 * 128)
    cost = pl.CostEstimate(
        flops=flops,
        transcendentals=n * (4 * 256 + 3),
        bytes_accessed=n * in_f * 4 + n * 3 * 4 + sum(
            a.size * 4 for a in scaled),
    )

    yt = pl.pallas_call(
        _siren_kernel,
        out_shape=jax.ShapeDtypeStruct((3, n), jnp.float32),
        grid=grid,
        in_specs=in_specs,
        out_specs=pl.BlockSpec((3, _TM), lambda i: (0, i)),
        compiler_params=pltpu.CompilerParams(
            dimension_semantics=("parallel",),
            vmem_limit_bytes=56 << 20,
        ),
        cost_estimate=cost,
    )(x, *scaled)
    return jnp.transpose(yt)


# speedup vs baseline: 7.8096x; 1.0666x over previous
import jax
import jax.numpy as jnp
from jax.experimental import pallas as pl
from jax.experimental.pallas import tpu as pltpu

_TM = 8192

_INV_2PI = 0.15915494309189535
_A = 1.3

_P0 = 4.833189964294434
_P1 = -29.079370498657227
_P2 = 44.6170768737793
_P3 = -22.523578643798828


def _sin_2pi(zt):
    k = jnp.rint(zt)
    u = zt - k
    t = jnp.tanh(_A * u)
    u2 = u * u
    p = (_P3 * u2 + _P2) * u2 + _P1
    p = p * u2 + _P0
    return t * p


_A2 = 1.86
_Q0 = 3.3808726170492873
_Q1 = -18.4928494080321
_Q2 = 19.88264853386534


def _sin_2pi_fast(zt):
    k = jnp.rint(zt)
    u = zt - k
    t = jnp.tanh(_A2 * u)
    u2 = u * u
    p = (_Q2 * u2 + _Q1) * u2 + _Q0
    return t * p


_C = float(_INV_2PI)


def _siren_kernel(x_ref, w0_ref, b0_ref, w1_ref, b1_ref, w2_ref, b2_ref,
                  w3_ref, b3_ref, w4_ref, b4_ref, o_ref):
    act = _sin_2pi(
        jnp.dot(x_ref[...], w0_ref[...], preferred_element_type=jnp.float32)
        * _C + b0_ref[...])
    for w_ref, b_ref, sin_fn in ((w1_ref, b1_ref, _sin_2pi),
                                 (w2_ref, b2_ref, _sin_2pi_fast),
                                 (w3_ref, b3_ref, _sin_2pi_fast)):
        z = jnp.dot(act, w_ref[...], preferred_element_type=jnp.float32)
        act = sin_fn(z * _C + b_ref[...])
    z4 = jnp.dot(act, w4_ref[...], preferred_element_type=jnp.float32)
    z4t = jnp.swapaxes(z4, 0, 1)[:3, :]
    o_ref[...] = _sin_2pi_fast(z4t * _C + b4_ref[...])


def kernel(x, w0, b0, w1, b1, w2, b2, w3, b3, w4, b4):
    n, in_f = x.shape
    grid = (pl.cdiv(n, _TM),)

    def _resident(shape):
        return pl.BlockSpec(shape, lambda i: (0,) * len(shape))

    c = jnp.float32(_INV_2PI)
    b4t = jnp.transpose(b4 * c)[:3, :]
    scaled = [w0, b0 * c, w1, b1 * c, w2, b2 * c, w3, b3 * c, w4, b4t]
    in_specs = [pl.BlockSpec((_TM, in_f), lambda i: (i, 0))]
    in_specs += [_resident(a.shape) for a in scaled]

    flops = 2 * n * (in_f * 256 + 3 * 256 * 256 + 256 * 128)
    cost = pl.CostEstimate(
        flops=flops,
        transcendentals=n * (4 * 256 + 3),
        bytes_accessed=n * in_f * 4 + n * 3 * 4 + sum(
            a.size * 4 for a in scaled),
    )

    yt = pl.pallas_call(
        _siren_kernel,
        out_shape=jax.ShapeDtypeStruct((3, n), jnp.float32),
        grid=grid,
        in_specs=in_specs,
        out_specs=pl.BlockSpec((3, _TM), lambda i: (0, i)),
        compiler_params=pltpu.CompilerParams(
            dimension_semantics=("parallel",),
            vmem_limit_bytes=56 << 20,
        ),
        cost_estimate=cost,
    )(x, *scaled)
    return jnp.transpose(yt)
